# baseline (speedup 1.0000x reference)
import functools

import numpy as np
import jax
import jax.numpy as jnp
from jax import lax
from jax.experimental import pallas as pl
from jax.experimental.pallas import tpu as pltpu
from jax.experimental.pallas import tpu_sc as plsc

_NC, _NS = 2, 16
_NW = _NC * _NS
_CHUNK = 128
_NBUF = 4
_L = 16

_BASE = 10000.0


def _cos_sin_tables(D, seq_end):
    inv_freq = 1.0 / (_BASE ** (np.arange(0, D, 2, dtype=np.float64) / D))
    t = np.arange(seq_end, dtype=np.float64)
    freqs = np.outer(t, inv_freq)
    freqs = np.concatenate((freqs, freqs), axis=-1)
    return (np.cos(freqs).astype(np.float32), np.sin(freqs).astype(np.float32))


_CT, _ST = _cos_sin_tables(16, 8)
_C64, _S64 = _cos_sin_tables(24, 64)

_CT_T = np.ascontiguousarray(_CT.T).reshape(-1)
_ST_T = np.ascontiguousarray(_ST.T).reshape(-1)
_C64_T = np.ascontiguousarray(_C64.T).reshape(-1)
_S64_T = np.ascontiguousarray(_S64.T).reshape(-1)


def _make_gather(N):
    assert N % (_NW * _CHUNK * _NBUF) == 0
    per_w = N // _NW
    n_chunks = per_w // _CHUNK
    n_outer = n_chunks // _NBUF
    n_grp = _CHUNK // _L
    mesh = plsc.VectorSubcoreMesh(core_axis_name="c", subcore_axis_name="s")
    f32 = jnp.float32

    @functools.partial(
        pl.kernel,
        mesh=mesh,
        compiler_params=pltpu.CompilerParams(
            use_tc_tiling_on_sc=False, needs_layout_passes=False),
        out_type=[
            jax.ShapeDtypeStruct((N * 16,), f32),
            jax.ShapeDtypeStruct((N * 16,), f32),
            jax.ShapeDtypeStruct((N * 24,), f32),
            jax.ShapeDtypeStruct((N * 24,), f32),
            jax.ShapeDtypeStruct((N * 24,), f32),
            jax.ShapeDtypeStruct((N * 24,), f32),
        ],
        scratch_types=(
            [pltpu.VMEM((_CHUNK,), jnp.int32) for _ in range(3 * _NBUF)]
            + [
                buf
                for _ in range(_NBUF)
                for buf in (
                    pltpu.VMEM((_CHUNK * 16,), f32),
                    pltpu.VMEM((_CHUNK * 16,), f32),
                    pltpu.VMEM((_CHUNK * 24,), f32),
                    pltpu.VMEM((_CHUNK * 24,), f32),
                    pltpu.VMEM((_CHUNK * 24,), f32),
                    pltpu.VMEM((_CHUNK * 24,), f32),
                )
            ]
            + [
                pltpu.VMEM((16 * 8,), f32),
                pltpu.VMEM((16 * 8,), f32),
                pltpu.VMEM((24 * 64,), f32),
                pltpu.VMEM((24 * 64,), f32),
            ]
            + [pltpu.SemaphoreType.DMA for _ in range(2 * _NBUF)]
        ),
    )
    def gather_kernel(pt, py, px, ct_h, st_h, c64_h, s64_h,
                      o_ct, o_st, o_cy, o_sy, o_cx, o_sx, *scratch):
        idx = [scratch[3 * s:3 * s + 3] for s in range(_NBUF)]
        rows = [scratch[3 * _NBUF + 6 * s:3 * _NBUF + 6 * s + 6]
                for s in range(_NBUF)]
        ct, st, c64, s64 = scratch[9 * _NBUF:9 * _NBUF + 4]
        sems = scratch[9 * _NBUF + 4:]
        semi = sems[0:_NBUF]
        semw = sems[_NBUF:2 * _NBUF]
        outs = (o_ct, o_st, o_cy, o_sy, o_cx, o_sx)
        out_d = (16, 16, 24, 24, 24, 24)
        pos = (pt, py, px)

        wid = lax.axis_index("s") * _NC + lax.axis_index("c")
        base = wid * per_w

        iota = lax.iota(jnp.int32, _L)
        scat_idx = {(d, r): iota * d + r for d in (16, 24) for r in range(8)}

        def issue_idx(s, c):
            tok0 = base + c * _CHUNK
            for p, ib in zip(pos, idx[s]):
                pltpu.async_copy(p.at[pl.ds(tok0, _CHUNK)], ib, semi[s])

        def wait_idx(s):
            for p, ib in zip(pos, idx[s]):
                pltpu.make_async_copy(p.at[pl.ds(0, _CHUNK)], ib, semi[s]).wait()

        def issue_writes(s, c):
            tok0 = base + c * _CHUNK
            for rb, o, d in zip(rows[s], outs, out_d):
                pltpu.async_copy(rb, o.at[pl.ds(tok0 * d, _CHUNK * d)], semw[s])

        def wait_writes(s):
            for rb, o, d in zip(rows[s], outs, out_d):
                pltpu.make_async_copy(rb, o.at[pl.ds(0, _CHUNK * d)], semw[s]).wait()

        def compute(s):
            it_r, iy_r, ix_r = idx[s]
            rct, rst, rcy, rsy, rcx, rsx = rows[s]

            def group_body(gi, carry):
                g0 = gi * _L
                for i_r, tabs_rbs, V, D in (
                    (it_r, ((ct, rct), (st, rst)), 8, 16),
                    (iy_r, ((c64, rcy), (s64, rsy)), 64, 24),
                    (ix_r, ((c64, rcx), (s64, rsx)), 64, 24),
                ):
                    iv = i_r[pl.ds(g0, _L)]
                    span = (_L - 1) * D + 8
                    for tab, rb in tabs_rbs:
                        for j in range(D):
                            vals = plsc.load_gather(tab, [iv + j * V])
                            plsc.store_scatter(
                                rb.at[pl.ds(g0 * D + (j // 8) * 8, span)],
                                [scat_idx[(D, j % 8)]], vals)
                return carry

            lax.fori_loop(0, n_grp, group_body, 0)

        for th, tv in zip((ct_h, st_h, c64_h, s64_h), (ct, st, c64, s64)):
            pltpu.sync_copy(th, tv)

        for s in range(_NBUF):
            issue_idx(s, s)

        def outer_body(g, carry):
            for k in range(_NBUF):
                s = k
                i = g * _NBUF + k
                wait_idx(s)

                @pl.when(g >= 1)
                def _():
                    wait_writes(s)

                compute(s)

                @pl.when(g < n_outer - 1)
                def _():
                    issue_idx(s, i + _NBUF)

                issue_writes(s, i)
            return carry

        lax.fori_loop(0, n_outer, outer_body, 0)
        for s in range(_NBUF):
            wait_writes(s)

    return gather_kernel


def kernel(dim, pos_t, pos_y, pos_x, max_t, max_y, max_x):
    ntok, B = pos_t.shape
    N = ntok * B
    pt = pos_t.reshape(N).astype(jnp.int32)
    py = pos_y.reshape(N).astype(jnp.int32)
    px = pos_x.reshape(N).astype(jnp.int32)
    tabs = (jnp.asarray(_CT_T), jnp.asarray(_ST_T),
            jnp.asarray(_C64_T), jnp.asarray(_S64_T))
    o_ct, o_st, o_cy, o_sy, o_cx, o_sx = _make_gather(N)(pt, py, px, *tabs)
    shp16 = (ntok, B, 1, 16)
    shp24 = (ntok, B, 1, 24)
    return (o_ct.reshape(shp16), o_st.reshape(shp16),
            o_cy.reshape(shp24), o_sy.reshape(shp24),
            o_cx.reshape(shp24), o_sx.reshape(shp24))


# speedup vs baseline: 1.1536x; 1.1536x over previous
import functools

import numpy as np
import jax
import jax.numpy as jnp
from jax import lax
from jax.experimental import pallas as pl
from jax.experimental.pallas import tpu as pltpu
from jax.experimental.pallas import tpu_sc as plsc

_NC, _NS = 2, 16
_NW = _NC * _NS
_CHUNK = 128
_NBUF = 4
_L = 16

_BASE = 10000.0


def _cos_sin_tables(D, seq_end):
    inv_freq = 1.0 / (_BASE ** (np.arange(0, D, 2, dtype=np.float64) / D))
    t = np.arange(seq_end, dtype=np.float64)
    freqs = np.outer(t, inv_freq)
    freqs = np.concatenate((freqs, freqs), axis=-1)
    return (np.cos(freqs).astype(np.float32), np.sin(freqs).astype(np.float32))


_CT, _ST = _cos_sin_tables(16, 8)
_C64, _S64 = _cos_sin_tables(24, 64)

_CT_T = np.ascontiguousarray(_CT[:, :8].T).reshape(-1)
_ST_T = np.ascontiguousarray(_ST[:, :8].T).reshape(-1)
_C64_T = np.ascontiguousarray(_C64[:, :12].T).reshape(-1)
_S64_T = np.ascontiguousarray(_S64[:, :12].T).reshape(-1)


def _make_gather(N):
    assert N % (_NW * _CHUNK * _NBUF) == 0
    per_w = N // _NW
    n_chunks = per_w // _CHUNK
    n_outer = n_chunks // _NBUF
    n_grp = _CHUNK // _L
    mesh = plsc.VectorSubcoreMesh(core_axis_name="c", subcore_axis_name="s")
    f32 = jnp.float32

    @functools.partial(
        pl.kernel,
        mesh=mesh,
        compiler_params=pltpu.CompilerParams(
            use_tc_tiling_on_sc=False, needs_layout_passes=False),
        out_type=[
            jax.ShapeDtypeStruct((N * 16,), f32),
            jax.ShapeDtypeStruct((N * 16,), f32),
            jax.ShapeDtypeStruct((N * 24,), f32),
            jax.ShapeDtypeStruct((N * 24,), f32),
            jax.ShapeDtypeStruct((N * 24,), f32),
            jax.ShapeDtypeStruct((N * 24,), f32),
        ],
        scratch_types=(
            [pltpu.VMEM((_CHUNK,), jnp.int32) for _ in range(3 * _NBUF)]
            + [
                buf
                for _ in range(_NBUF)
                for buf in (
                    pltpu.VMEM((_CHUNK * 16,), f32),
                    pltpu.VMEM((_CHUNK * 16,), f32),
                    pltpu.VMEM((_CHUNK * 24,), f32),
                    pltpu.VMEM((_CHUNK * 24,), f32),
                    pltpu.VMEM((_CHUNK * 24,), f32),
                    pltpu.VMEM((_CHUNK * 24,), f32),
                )
            ]
            + [
                pltpu.VMEM((8 * 8,), f32),
                pltpu.VMEM((8 * 8,), f32),
                pltpu.VMEM((12 * 64,), f32),
                pltpu.VMEM((12 * 64,), f32),
            ]
            + [pltpu.SemaphoreType.DMA for _ in range(2 * _NBUF)]
        ),
    )
    def gather_kernel(pt, py, px, ct_h, st_h, c64_h, s64_h,
                      o_ct, o_st, o_cy, o_sy, o_cx, o_sx, *scratch):
        idx = [scratch[3 * s:3 * s + 3] for s in range(_NBUF)]
        rows = [scratch[3 * _NBUF + 6 * s:3 * _NBUF + 6 * s + 6]
                for s in range(_NBUF)]
        ct, st, c64, s64 = scratch[9 * _NBUF:9 * _NBUF + 4]
        sems = scratch[9 * _NBUF + 4:]
        semi = sems[0:_NBUF]
        semw = sems[_NBUF:2 * _NBUF]
        outs = (o_ct, o_st, o_cy, o_sy, o_cx, o_sx)
        out_d = (16, 16, 24, 24, 24, 24)
        pos = (pt, py, px)

        wid = lax.axis_index("s") * _NC + lax.axis_index("c")
        base = wid * per_w

        iota = lax.iota(jnp.int32, _L)
        scat_idx = {(d, r): iota * d + r for d in (16, 24) for r in range(8)}

        def issue_idx(s, c):
            tok0 = base + c * _CHUNK
            for p, ib in zip(pos, idx[s]):
                pltpu.async_copy(p.at[pl.ds(tok0, _CHUNK)], ib, semi[s])

        def wait_idx(s):
            for p, ib in zip(pos, idx[s]):
                pltpu.make_async_copy(p.at[pl.ds(0, _CHUNK)], ib, semi[s]).wait()

        def issue_writes(s, c):
            tok0 = base + c * _CHUNK
            for rb, o, d in zip(rows[s], outs, out_d):
                pltpu.async_copy(rb, o.at[pl.ds(tok0 * d, _CHUNK * d)], semw[s])

        def wait_writes(s):
            for rb, o, d in zip(rows[s], outs, out_d):
                pltpu.make_async_copy(rb, o.at[pl.ds(0, _CHUNK * d)], semw[s]).wait()

        def compute(s):
            it_r, iy_r, ix_r = idx[s]
            rct, rst, rcy, rsy, rcx, rsx = rows[s]

            @plsc.parallel_loop(0, n_grp, unroll=2)
            def group_body(gi):
                g0 = gi * _L
                for i_r, ctab, stab, rbc, rbs, V, D in (
                    (it_r, ct, st, rct, rst, 8, 16),
                    (iy_r, c64, s64, rcy, rsy, 64, 24),
                    (ix_r, c64, s64, rcx, rsx, 64, 24),
                ):
                    half = D // 2
                    span = (_L - 1) * D + 8
                    iv = i_r[pl.ds(g0, _L)]
                    idxs = [iv + j * V for j in range(half)]
                    for tab, rb in ((ctab, rbc), (stab, rbs)):
                        vals = [plsc.load_gather(tab, [ix]) for ix in idxs]
                        for j in range(half):
                            for jj in (j, j + half):
                                plsc.store_scatter(
                                    rb.at[pl.ds(g0 * D + (jj // 8) * 8, span)],
                                    [scat_idx[(D, jj % 8)]], vals[j])

        for th, tv in zip((ct_h, st_h, c64_h, s64_h), (ct, st, c64, s64)):
            pltpu.sync_copy(th, tv)

        for s in range(_NBUF):
            issue_idx(s, s)

        def outer_body(g, carry):
            for k in range(_NBUF):
                s = k
                i = g * _NBUF + k
                wait_idx(s)

                @pl.when(g >= 1)
                def _():
                    wait_writes(s)

                compute(s)

                @pl.when(g < n_outer - 1)
                def _():
                    issue_idx(s, i + _NBUF)

                issue_writes(s, i)
            return carry

        lax.fori_loop(0, n_outer, outer_body, 0)
        for s in range(_NBUF):
            wait_writes(s)

    return gather_kernel


def kernel(dim, pos_t, pos_y, pos_x, max_t, max_y, max_x):
    ntok, B = pos_t.shape
    N = ntok * B
    pt = pos_t.reshape(N).astype(jnp.int32)
    py = pos_y.reshape(N).astype(jnp.int32)
    px = pos_x.reshape(N).astype(jnp.int32)
    tabs = (jnp.asarray(_CT_T), jnp.asarray(_ST_T),
            jnp.asarray(_C64_T), jnp.asarray(_S64_T))
    o_ct, o_st, o_cy, o_sy, o_cx, o_sx = _make_gather(N)(pt, py, px, *tabs)
    shp16 = (ntok, B, 1, 16)
    shp24 = (ntok, B, 1, 24)
    return (o_ct.reshape(shp16), o_st.reshape(shp16),
            o_cy.reshape(shp24), o_sy.reshape(shp24),
            o_cx.reshape(shp24), o_sx.reshape(shp24))


# speedup vs baseline: 4.9770x; 4.3143x over previous
import functools

import numpy as np
import jax
import jax.numpy as jnp
from jax import lax
from jax.experimental import pallas as pl
from jax.experimental.pallas import tpu as pltpu
from jax.experimental.pallas import tpu_sc as plsc

_NC, _NS = 2, 16
_NW = _NC * _NS
_CN = 64
_NBUF = 2
_L = 16

_BASE = 10000.0


def _cos_sin_tables(D, seq_end):
    inv_freq = 1.0 / (_BASE ** (np.arange(0, D, 2, dtype=np.float64) / D))
    t = np.arange(seq_end, dtype=np.float64)
    freqs = np.outer(t, inv_freq)
    freqs = np.concatenate((freqs, freqs), axis=-1)
    return (np.cos(freqs).astype(np.float32), np.sin(freqs).astype(np.float32))


_CT, _ST = _cos_sin_tables(16, 8)
_C64, _S64 = _cos_sin_tables(24, 64)

_CT_T = np.ascontiguousarray(_CT[:, :8].T).reshape(-1)
_ST_T = np.ascontiguousarray(_ST[:, :8].T).reshape(-1)
_C64_T = np.ascontiguousarray(_C64[:, :12].T).reshape(-1)
_S64_T = np.ascontiguousarray(_S64[:, :12].T).reshape(-1)


def _make_gather(ntok, B):
    assert B == 4 and ntok % (_NW * _CN * _NBUF) == 0
    n_tc = ntok // 128
    per_w = ntok // _NW // _CN
    n_outer = per_w // _NBUF
    n_grp = _CN // _L
    mesh = plsc.VectorSubcoreMesh(core_axis_name="c", subcore_axis_name="s")
    f32 = jnp.float32

    @functools.partial(
        pl.kernel,
        mesh=mesh,
        compiler_params=pltpu.CompilerParams(
            use_tc_tiling_on_sc=False, needs_layout_passes=False),
        out_type=[
            jax.ShapeDtypeStruct((B, 2, n_tc, 8, 128), f32),
            jax.ShapeDtypeStruct((B, 2, n_tc, 8, 128), f32),
            jax.ShapeDtypeStruct((B, 3, n_tc, 8, 128), f32),
            jax.ShapeDtypeStruct((B, 3, n_tc, 8, 128), f32),
            jax.ShapeDtypeStruct((B, 3, n_tc, 8, 128), f32),
            jax.ShapeDtypeStruct((B, 3, n_tc, 8, 128), f32),
        ],
        scratch_types=(
            [pltpu.VMEM((_CN * B,), jnp.int32) for _ in range(3 * _NBUF)]
            + [
                buf
                for _ in range(_NBUF)
                for buf in (
                    pltpu.VMEM((B, 2, 1, 8, _CN), f32),
                    pltpu.VMEM((B, 2, 1, 8, _CN), f32),
                    pltpu.VMEM((B, 3, 1, 8, _CN), f32),
                    pltpu.VMEM((B, 3, 1, 8, _CN), f32),
                    pltpu.VMEM((B, 3, 1, 8, _CN), f32),
                    pltpu.VMEM((B, 3, 1, 8, _CN), f32),
                )
            ]
            + [
                pltpu.VMEM((8 * 8,), f32),
                pltpu.VMEM((8 * 8,), f32),
                pltpu.VMEM((12 * 64,), f32),
                pltpu.VMEM((12 * 64,), f32),
            ]
            + [pltpu.SemaphoreType.DMA for _ in range(2 * _NBUF)]
        ),
    )
    def gather_kernel(pt, py, px, ct_h, st_h, c64_h, s64_h,
                      o_ct, o_st, o_cy, o_sy, o_cx, o_sx, *scratch):
        idx = [scratch[3 * s:3 * s + 3] for s in range(_NBUF)]
        rows = [scratch[3 * _NBUF + 6 * s:3 * _NBUF + 6 * s + 6]
                for s in range(_NBUF)]
        ct, st, c64, s64 = scratch[9 * _NBUF:9 * _NBUF + 4]
        sems = scratch[9 * _NBUF + 4:]
        semi = sems[0:_NBUF]
        semw = sems[_NBUF:2 * _NBUF]
        outs = (o_ct, o_st, o_cy, o_sy, o_cx, o_sx)
        pos = (pt, py, px)

        wid = lax.axis_index("s") * _NC + lax.axis_index("c")
        base = wid * per_w

        iota4 = lax.iota(jnp.int32, _L) * 4

        def issue_idx(s, c):
            tok0 = (base + c) * _CN * B
            for p, ib in zip(pos, idx[s]):
                pltpu.async_copy(p.at[pl.ds(tok0, _CN * B)], ib, semi[s])

        def wait_idx(s):
            for p, ib in zip(pos, idx[s]):
                pltpu.make_async_copy(p.at[pl.ds(0, _CN * B)], ib, semi[s]).wait()

        def issue_writes(s, c):
            ht = base + c
            tc = ht // 2
            h0 = (ht % 2) * _CN
            for rb, o in zip(rows[s], outs):
                pltpu.async_copy(
                    rb, o.at[:, :, pl.ds(tc, 1), :, pl.ds(h0, _CN)], semw[s])

        def wait_writes(s):
            for rb, o in zip(rows[s], outs):
                pltpu.make_async_copy(
                    rb, o.at[:, :, pl.ds(0, 1), :, pl.ds(0, _CN)], semw[s]).wait()

        def compute(s):
            it_r, iy_r, ix_r = idx[s]
            rct, rst, rcy, rsy, rcx, rsx = rows[s]

            def group_body(g, carry):
                g0 = g * _L
                for b in range(B):
                    lane_sel = iota4 + (g0 * B + b)
                    for i_r, ctab, stab, rbc, rbs, V, D in (
                        (it_r, ct, st, rct, rst, 8, 16),
                        (iy_r, c64, s64, rcy, rsy, 64, 24),
                        (ix_r, c64, s64, rcx, rsx, 64, 24),
                    ):
                        half = D // 2
                        iv = plsc.load_gather(i_r, [lane_sel])
                        for tab, rb in ((ctab, rbc), (stab, rbs)):
                            vals = [plsc.load_gather(tab, [iv + j * V])
                                    for j in range(half)]
                            for d in range(D):
                                rb[b, d // 8, 0, d % 8, pl.ds(g0, _L)] = (
                                    vals[d % half])
                return carry

            lax.fori_loop(0, n_grp, group_body, 0)

        for th, tv in zip((ct_h, st_h, c64_h, s64_h), (ct, st, c64, s64)):
            pltpu.sync_copy(th, tv)

        for s in range(_NBUF):
            issue_idx(s, s)

        def outer_body(g, carry):
            for k in range(_NBUF):
                s = k
                i = g * _NBUF + k
                wait_idx(s)

                @pl.when(g >= 1)
                def _():
                    wait_writes(s)

                compute(s)

                @pl.when(g < n_outer - 1)
                def _():
                    issue_idx(s, i + _NBUF)

                issue_writes(s, i)
            return carry

        lax.fori_loop(0, n_outer, outer_body, 0)
        for s in range(_NBUF):
            wait_writes(s)

    return gather_kernel


def kernel(dim, pos_t, pos_y, pos_x, max_t, max_y, max_x):
    ntok, B = pos_t.shape
    pt = pos_t.reshape(-1).astype(jnp.int32)
    py = pos_y.reshape(-1).astype(jnp.int32)
    px = pos_x.reshape(-1).astype(jnp.int32)
    tabs = (jnp.asarray(_CT_T), jnp.asarray(_ST_T),
            jnp.asarray(_C64_T), jnp.asarray(_S64_T))
    outs = _make_gather(ntok, B)(pt, py, px, *tabs)

    def unpack(o, d):
        return o.transpose(2, 4, 0, 1, 3).reshape(ntok, B, 1, d)

    o_ct, o_st, o_cy, o_sy, o_cx, o_sx = outs
    return (unpack(o_ct, 16), unpack(o_st, 16),
            unpack(o_cy, 24), unpack(o_sy, 24),
            unpack(o_cx, 24), unpack(o_sx, 24))


# speedup vs baseline: 9.5722x; 1.9233x over previous
import functools

import numpy as np
import jax
import jax.numpy as jnp
from jax import lax
from jax.experimental import pallas as pl
from jax.experimental.pallas import tpu as pltpu
from jax.experimental.pallas import tpu_sc as plsc

_NC, _NS = 2, 16
_NW = _NC * _NS
_CN = 64
_NBUF = 2
_L = 16

_BASE = 10000.0


def _cos_sin_tables(D, seq_end):
    inv_freq = 1.0 / (_BASE ** (np.arange(0, D, 2, dtype=np.float64) / D))
    t = np.arange(seq_end, dtype=np.float64)
    freqs = np.outer(t, inv_freq)
    freqs = np.concatenate((freqs, freqs), axis=-1)
    return (np.cos(freqs).astype(np.float32), np.sin(freqs).astype(np.float32))


_CT, _ST = _cos_sin_tables(16, 8)
_C64, _S64 = _cos_sin_tables(24, 64)

_CT_T = np.ascontiguousarray(_CT[:, :8].T).reshape(-1)
_ST_T = np.ascontiguousarray(_ST[:, :8].T).reshape(-1)
_C64_T = np.ascontiguousarray(_C64[:, :12].T).reshape(-1)
_S64_T = np.ascontiguousarray(_S64[:, :12].T).reshape(-1)


def _make_gather(ntok, B):
    assert B == 4 and ntok % (_NW * _CN * _NBUF) == 0
    n_tc = ntok // 128
    per_w = ntok // _NW // _CN
    n_outer = per_w // _NBUF
    n_grp = _CN // _L
    mesh = plsc.VectorSubcoreMesh(core_axis_name="c", subcore_axis_name="s")
    f32 = jnp.float32

    @functools.partial(
        pl.kernel,
        mesh=mesh,
        compiler_params=pltpu.CompilerParams(
            use_tc_tiling_on_sc=False, needs_layout_passes=False),
        out_type=[
            jax.ShapeDtypeStruct((B, 2, n_tc, 8, 128), f32),
            jax.ShapeDtypeStruct((B, 2, n_tc, 8, 128), f32),
            jax.ShapeDtypeStruct((B, 3, n_tc, 8, 128), f32),
            jax.ShapeDtypeStruct((B, 3, n_tc, 8, 128), f32),
            jax.ShapeDtypeStruct((B, 3, n_tc, 8, 128), f32),
            jax.ShapeDtypeStruct((B, 3, n_tc, 8, 128), f32),
        ],
        scratch_types=(
            [pltpu.VMEM((3, B, _CN), jnp.int32) for _ in range(_NBUF)]
            + [
                buf
                for _ in range(_NBUF)
                for buf in (
                    pltpu.VMEM((B, 2, 1, 8, _CN), f32),
                    pltpu.VMEM((B, 2, 1, 8, _CN), f32),
                    pltpu.VMEM((B, 3, 1, 8, _CN), f32),
                    pltpu.VMEM((B, 3, 1, 8, _CN), f32),
                    pltpu.VMEM((B, 3, 1, 8, _CN), f32),
                    pltpu.VMEM((B, 3, 1, 8, _CN), f32),
                )
            ]
            + [
                pltpu.VMEM((8 * 8,), f32),
                pltpu.VMEM((8 * 8,), f32),
                pltpu.VMEM((12 * 64,), f32),
                pltpu.VMEM((12 * 64,), f32),
            ]
            + [pltpu.SemaphoreType.DMA for _ in range(2 * _NBUF)]
        ),
    )
    def gather_kernel(pos3, ct_h, st_h, c64_h, s64_h,
                      o_ct, o_st, o_cy, o_sy, o_cx, o_sx, *scratch):
        idx = scratch[0:_NBUF]
        rows = [scratch[_NBUF + 6 * s:_NBUF + 6 * s + 6]
                for s in range(_NBUF)]
        ct, st, c64, s64 = scratch[7 * _NBUF:7 * _NBUF + 4]
        sems = scratch[7 * _NBUF + 4:]
        semi = sems[0:_NBUF]
        semw = sems[_NBUF:2 * _NBUF]
        outs = (o_ct, o_st, o_cy, o_sy, o_cx, o_sx)

        wid = lax.axis_index("s") * _NC + lax.axis_index("c")
        base = wid * per_w

        def issue_idx(s, c):
            n0 = (base + c) * _CN
            pltpu.async_copy(pos3.at[:, :, pl.ds(n0, _CN)], idx[s], semi[s])

        def wait_idx(s):
            pltpu.make_async_copy(
                pos3.at[:, :, pl.ds(0, _CN)], idx[s], semi[s]).wait()

        def issue_writes(s, c):
            ht = base + c
            tc = ht // 2
            h0 = (ht % 2) * _CN
            for rb, o in zip(rows[s], outs):
                pltpu.async_copy(
                    rb, o.at[:, :, pl.ds(tc, 1), :, pl.ds(h0, _CN)], semw[s])

        def wait_writes(s):
            for rb, o in zip(rows[s], outs):
                pltpu.make_async_copy(
                    rb, o.at[:, :, pl.ds(0, 1), :, pl.ds(0, _CN)], semw[s]).wait()

        def compute(s):
            ib = idx[s]
            rct, rst, rcy, rsy, rcx, rsx = rows[s]

            def group_body(g, carry):
                g0 = g * _L
                for b in range(B):
                    for a, ctab, stab, rbc, rbs, V, D in (
                        (0, ct, st, rct, rst, 8, 16),
                        (1, c64, s64, rcy, rsy, 64, 24),
                        (2, c64, s64, rcx, rsx, 64, 24),
                    ):
                        half = D // 2
                        iv = ib[a, b, pl.ds(g0, _L)]
                        for tab, rb in ((ctab, rbc), (stab, rbs)):
                            vals = [plsc.load_gather(tab, [iv + j * V])
                                    for j in range(half)]
                            for d in range(D):
                                rb[b, d // 8, 0, d % 8, pl.ds(g0, _L)] = (
                                    vals[d % half])
                return carry

            lax.fori_loop(0, n_grp, group_body, 0)

        for th, tv in zip((ct_h, st_h, c64_h, s64_h), (ct, st, c64, s64)):
            pltpu.sync_copy(th, tv)

        for s in range(_NBUF):
            issue_idx(s, s)

        def outer_body(g, carry):
            for k in range(_NBUF):
                s = k
                i = g * _NBUF + k
                wait_idx(s)

                @pl.when(g >= 1)
                def _():
                    wait_writes(s)

                compute(s)

                @pl.when(g < n_outer - 1)
                def _():
                    issue_idx(s, i + _NBUF)

                issue_writes(s, i)
            return carry

        lax.fori_loop(0, n_outer, outer_body, 0)
        for s in range(_NBUF):
            wait_writes(s)

    return gather_kernel


def kernel(dim, pos_t, pos_y, pos_x, max_t, max_y, max_x):
    ntok, B = pos_t.shape
    pos3 = jnp.stack([pos_t.T, pos_y.T, pos_x.T]).astype(jnp.int32)
    tabs = (jnp.asarray(_CT_T), jnp.asarray(_ST_T),
            jnp.asarray(_C64_T), jnp.asarray(_S64_T))
    outs = _make_gather(ntok, B)(pos3, *tabs)

    def unpack(o, d):
        return o.transpose(2, 4, 0, 1, 3).reshape(ntok, B, 1, d)

    o_ct, o_st, o_cy, o_sy, o_cx, o_sx = outs
    return (unpack(o_ct, 16), unpack(o_st, 16),
            unpack(o_cy, 24), unpack(o_sy, 24),
            unpack(o_cx, 24), unpack(o_sx, 24))


# speedup vs baseline: 10.6164x; 1.1091x over previous
import functools

import numpy as np
import jax
import jax.numpy as jnp
from jax import lax
from jax.experimental import pallas as pl
from jax.experimental.pallas import tpu as pltpu
from jax.experimental.pallas import tpu_sc as plsc

_NC, _NS = 2, 16
_NW = _NC * _NS
_CN = 128
_NBUF = 2
_L = 16

_BASE = 10000.0


def _cos_sin_tables(D, seq_end):
    inv_freq = 1.0 / (_BASE ** (np.arange(0, D, 2, dtype=np.float64) / D))
    t = np.arange(seq_end, dtype=np.float64)
    freqs = np.outer(t, inv_freq)
    freqs = np.concatenate((freqs, freqs), axis=-1)
    return (np.cos(freqs).astype(np.float32), np.sin(freqs).astype(np.float32))


_CT, _ST = _cos_sin_tables(16, 8)
_C64, _S64 = _cos_sin_tables(24, 64)

_CT_T = np.ascontiguousarray(_CT[:, :8].T).reshape(-1)
_ST_T = np.ascontiguousarray(_ST[:, :8].T).reshape(-1)
_C64_T = np.ascontiguousarray(_C64[:, :12].T).reshape(-1)
_S64_T = np.ascontiguousarray(_S64[:, :12].T).reshape(-1)


def _make_gather(ntok, B):
    assert B == 4 and ntok % (_NW * _CN * _NBUF) == 0
    n_tc = ntok // 128
    per_w = ntok // _NW // _CN
    n_outer = per_w // _NBUF
    n_grp = _CN // _L
    mesh = plsc.VectorSubcoreMesh(core_axis_name="c", subcore_axis_name="s")
    f32 = jnp.float32

    @functools.partial(
        pl.kernel,
        mesh=mesh,
        compiler_params=pltpu.CompilerParams(
            use_tc_tiling_on_sc=False, needs_layout_passes=False),
        out_type=[
            jax.ShapeDtypeStruct((B, 2, n_tc, 8, 128), f32),
            jax.ShapeDtypeStruct((B, 2, n_tc, 8, 128), f32),
            jax.ShapeDtypeStruct((B, 3, n_tc, 8, 128), f32),
            jax.ShapeDtypeStruct((B, 3, n_tc, 8, 128), f32),
            jax.ShapeDtypeStruct((B, 3, n_tc, 8, 128), f32),
            jax.ShapeDtypeStruct((B, 3, n_tc, 8, 128), f32),
        ],
        scratch_types=(
            [pltpu.VMEM((3, B, _CN), jnp.int32) for _ in range(_NBUF)]
            + [
                buf
                for _ in range(_NBUF)
                for buf in (
                    pltpu.VMEM((B, 1, 1, 8, _CN), f32),
                    pltpu.VMEM((B, 1, 1, 8, _CN), f32),
                    pltpu.VMEM((B, 1, 1, 12, _CN), f32),
                    pltpu.VMEM((B, 1, 1, 12, _CN), f32),
                    pltpu.VMEM((B, 1, 1, 12, _CN), f32),
                    pltpu.VMEM((B, 1, 1, 12, _CN), f32),
                )
            ]
            + [
                pltpu.VMEM((8 * 8,), f32),
                pltpu.VMEM((8 * 8,), f32),
                pltpu.VMEM((12 * 64,), f32),
                pltpu.VMEM((12 * 64,), f32),
            ]
            + [pltpu.SemaphoreType.DMA for _ in range(2 * _NBUF)]
        ),
    )
    def gather_kernel(pos3, ct_h, st_h, c64_h, s64_h,
                      o_ct, o_st, o_cy, o_sy, o_cx, o_sx, *scratch):
        idx = scratch[0:_NBUF]
        rows = [scratch[_NBUF + 6 * s:_NBUF + 6 * s + 6]
                for s in range(_NBUF)]
        ct, st, c64, s64 = scratch[7 * _NBUF:7 * _NBUF + 4]
        sems = scratch[7 * _NBUF + 4:]
        semi = sems[0:_NBUF]
        semw = sems[_NBUF:2 * _NBUF]
        outs = (o_ct, o_st, o_cy, o_sy, o_cx, o_sx)

        wid = lax.axis_index("s") * _NC + lax.axis_index("c")
        base = wid * per_w

        def issue_idx(s, c):
            n0 = (base + c) * _CN
            pltpu.async_copy(pos3.at[:, :, pl.ds(n0, _CN)], idx[s], semi[s])

        def wait_idx(s):
            pltpu.make_async_copy(
                pos3.at[:, :, pl.ds(0, _CN)], idx[s], semi[s]).wait()

        _dup16 = ((0, 0, 0, 8), (1, 0, 0, 8))
        _dup24 = ((0, 0, 0, 8), (1, 0, 8, 4), (1, 4, 0, 4), (2, 0, 4, 8))
        out_dup = (_dup16, _dup16, _dup24, _dup24, _dup24, _dup24)

        def issue_writes(s, c):
            tc = base + c
            for rb, o, dup in zip(rows[s], outs, out_dup):
                for tr, r0, sr0, ln in dup:
                    pltpu.async_copy(
                        rb.at[:, :, :, pl.ds(sr0, ln), :],
                        o.at[:, pl.ds(tr, 1), pl.ds(tc, 1), pl.ds(r0, ln), :],
                        semw[s])

        def wait_writes(s):
            for rb, o, dup in zip(rows[s], outs, out_dup):
                for tr, r0, sr0, ln in dup:
                    pltpu.make_async_copy(
                        rb.at[:, :, :, pl.ds(sr0, ln), :],
                        o.at[:, pl.ds(tr, 1), pl.ds(0, 1), pl.ds(r0, ln), :],
                        semw[s]).wait()

        def compute(s):
            ib = idx[s]
            rct, rst, rcy, rsy, rcx, rsx = rows[s]

            @plsc.parallel_loop(0, n_grp, unroll=2)
            def group_body(g):
                g0 = g * _L
                for b in range(B):
                    for a, ctab, stab, rbc, rbs, V, D in (
                        (0, ct, st, rct, rst, 8, 16),
                        (1, c64, s64, rcy, rsy, 64, 24),
                        (2, c64, s64, rcx, rsx, 64, 24),
                    ):
                        half = D // 2
                        iv = ib[a, b, pl.ds(g0, _L)]
                        for tab, rb in ((ctab, rbc), (stab, rbs)):
                            vals = [plsc.load_gather(tab, [iv + j * V])
                                    for j in range(half)]
                            for j in range(half):
                                rb[b, 0, 0, j, pl.ds(g0, _L)] = vals[j]

        for th, tv in zip((ct_h, st_h, c64_h, s64_h), (ct, st, c64, s64)):
            pltpu.sync_copy(th, tv)

        for s in range(_NBUF):
            issue_idx(s, s)

        def outer_body(g, carry):
            for k in range(_NBUF):
                s = k
                i = g * _NBUF + k
                wait_idx(s)

                @pl.when(g >= 1)
                def _():
                    wait_writes(s)

                compute(s)

                @pl.when(g < n_outer - 1)
                def _():
                    issue_idx(s, i + _NBUF)

                issue_writes(s, i)
            return carry

        lax.fori_loop(0, n_outer, outer_body, 0)
        for s in range(_NBUF):
            wait_writes(s)

    return gather_kernel


def kernel(dim, pos_t, pos_y, pos_x, max_t, max_y, max_x):
    ntok, B = pos_t.shape
    pos3 = jnp.stack([pos_t.T, pos_y.T, pos_x.T]).astype(jnp.int32)
    tabs = (jnp.asarray(_CT_T), jnp.asarray(_ST_T),
            jnp.asarray(_C64_T), jnp.asarray(_S64_T))
    outs = _make_gather(ntok, B)(pos3, *tabs)

    def unpack(o, d):
        return o.transpose(2, 4, 0, 1, 3).reshape(ntok, B, 1, d)

    o_ct, o_st, o_cy, o_sy, o_cx, o_sx = outs
    return (unpack(o_ct, 16), unpack(o_st, 16),
            unpack(o_cy, 24), unpack(o_sy, 24),
            unpack(o_cx, 24), unpack(o_sx, 24))
